# Initial kernel scaffold; baseline (speedup 1.0000x reference)
import jax
import jax.numpy as jnp
from jax.experimental import pallas as pl


def kernel(x, edge_index, batch, W, b, att_src, att_dst):
    raise NotImplementedError("write your pallas kernel here")


# speedup vs baseline: 62.8008x; 62.8008x over previous
import functools

import jax
import jax.numpy as jnp
from jax import lax
from jax.experimental import pallas as pl
from jax.experimental.pallas import tpu as pltpu
from jax.experimental.pallas import tpu_sc as plsc

NC, NS, L = 2, 16, 16
NW = NC * NS
NEG_SLOPE = 0.2
NUM_GRAPHS = 512


def _pass_a(x, w_row, nrows, n):
    d = x.shape[1]

    def body(wt_ref, x_ref, h_ref, hmax_ref, hmin_ref):
        i = pl.program_id(0)
        xb = x_ref[...]
        hrow = lax.dot_general(wt_ref[...], xb, (((1,), (1,)), ((), ())),
                               preferred_element_type=jnp.float32)
        lane = lax.broadcasted_iota(jnp.int32, (1, 128), 1)
        valid = (i * 128 + lane) < n
        hrow = jnp.where(valid, hrow, 0.0)
        h_ref[0] = hrow
        bmax = jnp.max(hrow)
        bmin = jnp.min(hrow)

        @pl.when(i == 0)
        def _():
            hmax_ref[0] = bmax
            hmin_ref[0] = bmin

        @pl.when(i > 0)
        def _():
            hmax_ref[0] = jnp.maximum(hmax_ref[0], bmax)
            hmin_ref[0] = jnp.minimum(hmin_ref[0], bmin)

    return pl.pallas_call(
        body,
        grid=(nrows,),
        in_specs=[pl.BlockSpec((1, d), lambda i: (0, 0)),
                  pl.BlockSpec((128, d), lambda i: (i, 0))],
        out_specs=[pl.BlockSpec((1, 1, 128), lambda i: (i, 0, 0)),
                   pl.BlockSpec(memory_space=pltpu.SMEM),
                   pl.BlockSpec(memory_space=pltpu.SMEM)],
        out_shape=[jax.ShapeDtypeStruct((nrows, 1, 128), jnp.float32),
                   jax.ShapeDtypeStruct((1,), jnp.float32),
                   jax.ShapeDtypeStruct((1,), jnp.float32)],
    )(w_row, x)


def _make_edge_kernel(num_edges, npad, chunk):
    nchunks = num_edges // chunk
    iters = -(-nchunks // NW)
    npw = npad // NS
    nvec = chunk // L
    mesh = plsc.VectorSubcoreMesh(core_axis_name="c", subcore_axis_name="s",
                                  num_cores=NC, num_subcores=NS)

    def body(ei, h_hbm, par_hbm, num_out, den_out,
             h_v, par_v, src_v, dst_v, e_v, eh_v, zero_v, num_sh, den_sh):
        c = lax.axis_index("c")
        s = lax.axis_index("s")
        w = c * NS + s

        def zbody(j, _):
            zero_v[pl.ds(j * L, L)] = jnp.zeros((L,), jnp.float32)
            return 0

        lax.fori_loop(0, npw // L, zbody, 0)
        pltpu.sync_copy(zero_v, num_sh.at[pl.ds(s * npw, npw)])
        pltpu.sync_copy(zero_v, den_sh.at[pl.ds(s * npw, npw)])
        pltpu.sync_copy(h_hbm, h_v)
        pltpu.sync_copy(par_hbm, par_v)
        plsc.subcore_barrier()
        ats = par_v[0]
        atd = par_v[1]
        cshift = par_v[2]

        def chunk_fn(cid):
            base = cid * chunk
            pltpu.sync_copy(ei.at[0, pl.ds(base, chunk)], src_v)
            pltpu.sync_copy(ei.at[1, pl.ds(base, chunk)], dst_v)

            def vbody(j, _):
                sidx = src_v[pl.ds(j * L, L)]
                didx = dst_v[pl.ds(j * L, L)]
                hs = plsc.load_gather(h_v, [sidx])
                hd = plsc.load_gather(h_v, [didx])
                z = ats * hs + atd * hd
                v = jnp.where(z >= 0, z, NEG_SLOPE * z)
                e = jnp.exp(v - cshift)
                e_v[pl.ds(j * L, L)] = e
                eh_v[pl.ds(j * L, L)] = e * hs
                return 0

            lax.fori_loop(0, nvec, vbody, 0)
            pltpu.sync_copy(e_v, den_sh.at[dst_v], add=True)
            pltpu.sync_copy(eh_v, num_sh.at[dst_v], add=True)

        def loop_body(i, _):
            cid = w + i * NW

            @pl.when(cid < nchunks)
            def _():
                chunk_fn(cid)

            return 0

        lax.fori_loop(0, iters, loop_body, 0)
        plsc.subcore_barrier()
        pltpu.sync_copy(num_sh.at[pl.ds(s * npw, npw)], zero_v)
        pltpu.sync_copy(zero_v, num_out.at[pl.ds(c * npad + s * npw, npw)])
        pltpu.sync_copy(den_sh.at[pl.ds(s * npw, npw)], zero_v)
        pltpu.sync_copy(zero_v, den_out.at[pl.ds(c * npad + s * npw, npw)])

    return pl.kernel(
        body,
        out_type=[jax.ShapeDtypeStruct((NC * npad,), jnp.float32),
                  jax.ShapeDtypeStruct((NC * npad,), jnp.float32)],
        mesh=mesh,
        compiler_params=pltpu.CompilerParams(needs_layout_passes=False),
        scratch_types=[
            pltpu.VMEM((npad,), jnp.float32),
            pltpu.VMEM((4, L), jnp.float32),
            pltpu.VMEM((chunk,), jnp.int32),
            pltpu.VMEM((chunk,), jnp.int32),
            pltpu.VMEM((chunk,), jnp.float32),
            pltpu.VMEM((chunk,), jnp.float32),
            pltpu.VMEM((npw,), jnp.float32),
            pltpu.VMEM_SHARED((npad,), jnp.float32),
            pltpu.VMEM_SHARED((npad,), jnp.float32),
        ],
    )


def _pass_c(par, x, h3, batch3, num4, den4, nrows, n, g):
    d = x.shape[1]

    def body(par_ref, x_ref, h_ref, bat_ref, n0_ref, n1_ref, d0_ref, d1_ref,
             gx_ref, dacc):
        i = pl.program_id(0)
        atsum = par_ref[0]
        csh = par_ref[1]
        bb = par_ref[2]
        m2 = par_ref[3]
        h = h_ref[0]
        n0 = n0_ref[0, 0]
        n1 = n1_ref[0, 0]
        d0 = d0_ref[0, 0]
        d1 = d1_ref[0, 0]
        z = atsum * h
        vv = jnp.where(z >= 0, z, NEG_SLOPE * z)
        es = jnp.exp(vv - csh)
        ntot = n0 + n1 + es * h
        dtot = d0 + d1 + es
        sval = ntot / dtot + bb
        e2 = jnp.exp(sval - m2)
        bat = bat_ref[0]
        gid = lax.broadcasted_iota(jnp.int32, (g, 1), 0)
        onehot_t = jnp.where(bat == gid, e2, 0.0)
        lane = lax.broadcasted_iota(jnp.int32, (128, 1), 0)
        rvalid = (i * 128 + lane) < n
        xb = jnp.where(rvalid, x_ref[...], 0.0)
        part = lax.dot_general(onehot_t, xb, (((1,), (0,)), ((), ())),
                               preferred_element_type=jnp.float32)
        partd = jnp.sum(onehot_t, axis=1, keepdims=True)

        @pl.when(i == 0)
        def _():
            gx_ref[...] = part
            dacc[...] = partd

        @pl.when(i > 0)
        def _():
            gx_ref[...] += part
            dacc[...] += partd

        @pl.when(i == nrows - 1)
        def _():
            gx_ref[...] = gx_ref[...] / (dacc[...] + 1e-16)

    return pl.pallas_call(
        body,
        grid=(nrows,),
        in_specs=[pl.BlockSpec(memory_space=pltpu.SMEM),
                  pl.BlockSpec((128, d), lambda i: (i, 0)),
                  pl.BlockSpec((1, 1, 128), lambda i: (i, 0, 0)),
                  pl.BlockSpec((1, 1, 128), lambda i: (i, 0, 0)),
                  pl.BlockSpec((1, 1, 1, 128), lambda i: (0, i, 0, 0)),
                  pl.BlockSpec((1, 1, 1, 128), lambda i: (1, i, 0, 0)),
                  pl.BlockSpec((1, 1, 1, 128), lambda i: (0, i, 0, 0)),
                  pl.BlockSpec((1, 1, 1, 128), lambda i: (1, i, 0, 0))],
        out_specs=pl.BlockSpec((g, d), lambda i: (0, 0)),
        out_shape=jax.ShapeDtypeStruct((g, d), jnp.float32),
        scratch_shapes=[pltpu.VMEM((g, 1), jnp.float32)],
    )(par, x, h3, batch3, num4, num4, den4, den4)


def kernel(x, edge_index, batch, W, b, att_src, att_dst):
    n, d = x.shape
    num_edges = edge_index.shape[1]
    g = NUM_GRAPHS
    nrows = -(-n // 128)
    npad = nrows * 128
    chunk = 2560

    w_row = W.reshape(1, d)
    h3, hmax, hmin = _pass_a(x, w_row, nrows, n)
    hmax_s = hmax[0]
    hmin_s = hmin[0]
    mas = jnp.where(att_src[0] >= 0, att_src[0] * hmax_s, att_src[0] * hmin_s)
    mad = jnp.where(att_dst[0] >= 0, att_dst[0] * hmax_s, att_dst[0] * hmin_s)
    amax = mas + mad
    cshift = jnp.where(amax >= 0, amax, NEG_SLOPE * amax)

    par_sc = jnp.stack([
        jnp.full((L,), att_src[0], jnp.float32),
        jnp.full((L,), att_dst[0], jnp.float32),
        jnp.full((L,), cshift, jnp.float32),
        jnp.zeros((L,), jnp.float32),
    ])
    h_flat = h3.reshape(npad)
    num2, den2 = _make_edge_kernel(num_edges, npad, chunk)(
        edge_index, h_flat, par_sc)

    m2 = hmax_s + b[0]
    par_tc = jnp.stack([att_src[0] + att_dst[0], cshift, b[0], m2])
    batch_p = jnp.concatenate(
        [batch, jnp.full((npad - n,), g, jnp.int32)]).reshape(nrows, 1, 128)
    num4 = num2.reshape(NC, nrows, 1, 128)
    den4 = den2.reshape(NC, nrows, 1, 128)
    return _pass_c(par_tc, x, h3, batch_p, num4, den4, nrows, n, g)


# speedup vs baseline: 201.3465x; 3.2061x over previous
import functools

import jax
import jax.numpy as jnp
from jax import lax
from jax.experimental import pallas as pl
from jax.experimental.pallas import tpu as pltpu
from jax.experimental.pallas import tpu_sc as plsc

NC, NS, L = 2, 16, 16
NW = NC * NS
NEG_SLOPE = 0.2
NUM_GRAPHS = 512


BR = 8


def _pass_a(x, w_row, nrows, n):
    d = x.shape[1]
    nb = -(-nrows // BR)

    def body(wt_ref, x_ref, h_ref, hmax_ref, hmin_ref):
        i = pl.program_id(0)
        xb = x_ref[...]
        wt = wt_ref[...]
        lane = lax.broadcasted_iota(jnp.int32, (1, 128), 1)
        bmax = None
        for r in range(BR):
            hrow = lax.dot_general(wt, xb[r * 128:(r + 1) * 128, :],
                                   (((1,), (1,)), ((), ())),
                                   preferred_element_type=jnp.float32)
            valid = ((i * BR + r) * 128 + lane) < n
            hrow = jnp.where(valid, hrow, 0.0)
            h_ref[r] = hrow
            rmax = jnp.max(hrow)
            rmin = jnp.min(hrow)
            bmax = rmax if bmax is None else jnp.maximum(bmax, rmax)
            bmin = rmin if r == 0 else jnp.minimum(bmin, rmin)

        @pl.when(i == 0)
        def _():
            hmax_ref[0] = bmax
            hmin_ref[0] = bmin

        @pl.when(i > 0)
        def _():
            hmax_ref[0] = jnp.maximum(hmax_ref[0], bmax)
            hmin_ref[0] = jnp.minimum(hmin_ref[0], bmin)

    return pl.pallas_call(
        body,
        grid=(nb,),
        in_specs=[pl.BlockSpec((1, d), lambda i: (0, 0)),
                  pl.BlockSpec((BR * 128, d), lambda i: (i, 0))],
        out_specs=[pl.BlockSpec((BR, 1, 128), lambda i: (i, 0, 0)),
                   pl.BlockSpec(memory_space=pltpu.SMEM),
                   pl.BlockSpec(memory_space=pltpu.SMEM)],
        out_shape=[jax.ShapeDtypeStruct((nrows, 1, 128), jnp.float32),
                   jax.ShapeDtypeStruct((1,), jnp.float32),
                   jax.ShapeDtypeStruct((1,), jnp.float32)],
    )(w_row, x)


def _make_edge_kernel(num_edges, npad, chunk):
    nchunks = num_edges // chunk
    iters = -(-nchunks // NW)
    npw = npad // NS
    nvec = chunk // L
    mesh = plsc.VectorSubcoreMesh(core_axis_name="c", subcore_axis_name="s",
                                  num_cores=NC, num_subcores=NS)

    def body(ei, h_hbm, par_hbm, num_out, den_out,
             h_v, par_v, src_v, dst_v, e_v, eh_v, zero_v, num_sh, den_sh):
        c = lax.axis_index("c")
        s = lax.axis_index("s")
        w = c * NS + s

        def zbody(j, _):
            zero_v[pl.ds(j * L, L)] = jnp.zeros((L,), jnp.float32)
            return 0

        lax.fori_loop(0, npw // L, zbody, 0)
        pltpu.sync_copy(zero_v, num_sh.at[pl.ds(s * npw, npw)])
        pltpu.sync_copy(zero_v, den_sh.at[pl.ds(s * npw, npw)])
        pltpu.sync_copy(h_hbm, h_v)
        pltpu.sync_copy(par_hbm, par_v)
        plsc.subcore_barrier()
        ats = par_v[0]
        atd = par_v[1]
        cshift = par_v[2]

        def chunk_fn(cid):
            base = cid * chunk
            pltpu.sync_copy(ei.at[0, pl.ds(base, chunk)], src_v)
            pltpu.sync_copy(ei.at[1, pl.ds(base, chunk)], dst_v)

            def vbody(j, _):
                sidx = src_v[pl.ds(j * L, L)]
                didx = dst_v[pl.ds(j * L, L)]
                hs = plsc.load_gather(h_v, [sidx])
                hd = plsc.load_gather(h_v, [didx])
                z = ats * hs + atd * hd
                v = jnp.where(z >= 0, z, NEG_SLOPE * z)
                e = jnp.exp(v - cshift)
                e_v[pl.ds(j * L, L)] = e
                eh_v[pl.ds(j * L, L)] = e * hs
                return 0

            lax.fori_loop(0, nvec, vbody, 0)
            pltpu.sync_copy(e_v, den_sh.at[dst_v], add=True)
            pltpu.sync_copy(eh_v, num_sh.at[dst_v], add=True)

        def loop_body(i, _):
            cid = w + i * NW

            @pl.when(cid < nchunks)
            def _():
                chunk_fn(cid)

            return 0

        lax.fori_loop(0, iters, loop_body, 0)
        plsc.subcore_barrier()
        pltpu.sync_copy(num_sh.at[pl.ds(s * npw, npw)], zero_v)
        pltpu.sync_copy(zero_v, num_out.at[pl.ds(c * npad + s * npw, npw)])
        pltpu.sync_copy(den_sh.at[pl.ds(s * npw, npw)], zero_v)
        pltpu.sync_copy(zero_v, den_out.at[pl.ds(c * npad + s * npw, npw)])

    return pl.kernel(
        body,
        out_type=[jax.ShapeDtypeStruct((NC * npad,), jnp.float32),
                  jax.ShapeDtypeStruct((NC * npad,), jnp.float32)],
        mesh=mesh,
        compiler_params=pltpu.CompilerParams(needs_layout_passes=False),
        scratch_types=[
            pltpu.VMEM((npad,), jnp.float32),
            pltpu.VMEM((4, L), jnp.float32),
            pltpu.VMEM((chunk,), jnp.int32),
            pltpu.VMEM((chunk,), jnp.int32),
            pltpu.VMEM((chunk,), jnp.float32),
            pltpu.VMEM((chunk,), jnp.float32),
            pltpu.VMEM((npw,), jnp.float32),
            pltpu.VMEM_SHARED((npad,), jnp.float32),
            pltpu.VMEM_SHARED((npad,), jnp.float32),
        ],
    )


def _pass_c(par, x, h3, batch3, num4, den4, nrows, n, g):
    d = x.shape[1]

    nb = -(-nrows // BR)

    def body(par_ref, x_ref, h_ref, bat_ref, n0_ref, n1_ref, d0_ref, d1_ref,
             gx_ref, dacc):
        i = pl.program_id(0)
        atsum = par_ref[0]
        csh = par_ref[1]
        bb = par_ref[2]
        m2 = par_ref[3]
        h = h_ref[:, 0, :]
        n0 = n0_ref[0, :, 0, :]
        n1 = n1_ref[0, :, 0, :]
        d0 = d0_ref[0, :, 0, :]
        d1 = d1_ref[0, :, 0, :]
        z = atsum * h
        vv = jnp.where(z >= 0, z, NEG_SLOPE * z)
        es = jnp.exp(vv - csh)
        ntot = n0 + n1 + es * h
        dtot = d0 + d1 + es
        sval = ntot / dtot + bb
        e2 = jnp.exp(sval - m2)
        lane = lax.broadcasted_iota(jnp.int32, (BR, 128), 1)
        row = lax.broadcasted_iota(jnp.int32, (BR, 128), 0)
        nid = (i * BR + row) * 128 + lane
        e2 = jnp.where(nid < n, e2, 0.0)
        bat = bat_ref[:, 0, :]
        gid = lax.broadcasted_iota(jnp.int32, (g, 1), 0)
        xrow = lax.broadcasted_iota(jnp.int32, (BR * 128, 1), 0)
        xb = jnp.where(i * BR * 128 + xrow < n, x_ref[...], 0.0)
        part = None
        partd = None
        for r in range(BR):
            onehot_t = jnp.where(bat[r:r + 1, :] == gid, e2[r:r + 1, :], 0.0)
            pr = lax.dot_general(onehot_t, xb[r * 128:(r + 1) * 128, :],
                                 (((1,), (0,)), ((), ())),
                                 preferred_element_type=jnp.float32)
            dr = jnp.sum(onehot_t, axis=1, keepdims=True)
            part = pr if part is None else part + pr
            partd = dr if partd is None else partd + dr

        @pl.when(i == 0)
        def _():
            gx_ref[...] = part
            dacc[...] = partd

        @pl.when(i > 0)
        def _():
            gx_ref[...] += part
            dacc[...] += partd

        @pl.when(i == nb - 1)
        def _():
            gx_ref[...] = gx_ref[...] / (dacc[...] + 1e-16)

    return pl.pallas_call(
        body,
        grid=(nb,),
        in_specs=[pl.BlockSpec(memory_space=pltpu.SMEM),
                  pl.BlockSpec((BR * 128, d), lambda i: (i, 0)),
                  pl.BlockSpec((BR, 1, 128), lambda i: (i, 0, 0)),
                  pl.BlockSpec((BR, 1, 128), lambda i: (i, 0, 0)),
                  pl.BlockSpec((1, BR, 1, 128), lambda i: (0, i, 0, 0)),
                  pl.BlockSpec((1, BR, 1, 128), lambda i: (1, i, 0, 0)),
                  pl.BlockSpec((1, BR, 1, 128), lambda i: (0, i, 0, 0)),
                  pl.BlockSpec((1, BR, 1, 128), lambda i: (1, i, 0, 0))],
        out_specs=pl.BlockSpec((g, d), lambda i: (0, 0)),
        out_shape=jax.ShapeDtypeStruct((g, d), jnp.float32),
        scratch_shapes=[pltpu.VMEM((g, 1), jnp.float32)],
    )(par, x, h3, batch3, num4, num4, den4, den4)


def kernel(x, edge_index, batch, W, b, att_src, att_dst):
    n, d = x.shape
    num_edges = edge_index.shape[1]
    g = NUM_GRAPHS
    nrows = -(-n // 128)
    npad = nrows * 128
    chunk = 2560

    w_row = W.reshape(1, d)
    h3, hmax, hmin = _pass_a(x, w_row, nrows, n)
    hmax_s = hmax[0]
    hmin_s = hmin[0]
    mas = jnp.where(att_src[0] >= 0, att_src[0] * hmax_s, att_src[0] * hmin_s)
    mad = jnp.where(att_dst[0] >= 0, att_dst[0] * hmax_s, att_dst[0] * hmin_s)
    amax = mas + mad
    cshift = jnp.where(amax >= 0, amax, NEG_SLOPE * amax)

    par_sc = jnp.stack([
        jnp.full((L,), att_src[0], jnp.float32),
        jnp.full((L,), att_dst[0], jnp.float32),
        jnp.full((L,), cshift, jnp.float32),
        jnp.zeros((L,), jnp.float32),
    ])
    h_flat = h3.reshape(npad)
    num2, den2 = _make_edge_kernel(num_edges, npad, chunk)(
        edge_index, h_flat, par_sc)

    m2 = hmax_s + b[0]
    par_tc = jnp.stack([att_src[0] + att_dst[0], cshift, b[0], m2])
    batch_p = jnp.concatenate(
        [batch, jnp.full((npad - n,), g, jnp.int32)]).reshape(nrows, 1, 128)
    num4 = num2.reshape(NC, nrows, 1, 128)
    den4 = den2.reshape(NC, nrows, 1, 128)
    return _pass_c(par_tc, x, h3, batch_p, num4, den4, nrows, n, g)


# speedup vs baseline: 202.7947x; 1.0072x over previous
import functools

import jax
import jax.numpy as jnp
from jax import lax
from jax.experimental import pallas as pl
from jax.experimental.pallas import tpu as pltpu
from jax.experimental.pallas import tpu_sc as plsc

NC, NS, L = 2, 16, 16
NW = NC * NS
NEG_SLOPE = 0.2
NUM_GRAPHS = 512


BR = 8


def _pass_a(x, w_row, nrows, n):
    d = x.shape[1]
    nb = -(-nrows // BR)

    def body(wt_ref, x_ref, h_ref, hmax_ref, hmin_ref):
        i = pl.program_id(0)
        xb = x_ref[...]
        wt = wt_ref[...]
        lane = lax.broadcasted_iota(jnp.int32, (1, 128), 1)
        bmax = None
        for r in range(BR):
            hrow = lax.dot_general(wt, xb[r * 128:(r + 1) * 128, :],
                                   (((1,), (1,)), ((), ())),
                                   preferred_element_type=jnp.float32)
            valid = ((i * BR + r) * 128 + lane) < n
            hrow = jnp.where(valid, hrow, 0.0)
            h_ref[r] = hrow
            rmax = jnp.max(hrow)
            rmin = jnp.min(hrow)
            bmax = rmax if bmax is None else jnp.maximum(bmax, rmax)
            bmin = rmin if r == 0 else jnp.minimum(bmin, rmin)

        @pl.when(i == 0)
        def _():
            hmax_ref[0] = bmax
            hmin_ref[0] = bmin

        @pl.when(i > 0)
        def _():
            hmax_ref[0] = jnp.maximum(hmax_ref[0], bmax)
            hmin_ref[0] = jnp.minimum(hmin_ref[0], bmin)

    return pl.pallas_call(
        body,
        grid=(nb,),
        in_specs=[pl.BlockSpec((1, d), lambda i: (0, 0)),
                  pl.BlockSpec((BR * 128, d), lambda i: (i, 0))],
        out_specs=[pl.BlockSpec((BR, 1, 128), lambda i: (i, 0, 0)),
                   pl.BlockSpec(memory_space=pltpu.SMEM),
                   pl.BlockSpec(memory_space=pltpu.SMEM)],
        out_shape=[jax.ShapeDtypeStruct((nrows, 1, 128), jnp.float32),
                   jax.ShapeDtypeStruct((1,), jnp.float32),
                   jax.ShapeDtypeStruct((1,), jnp.float32)],
    )(w_row, x)


def _make_edge_kernel(num_edges, npad, chunk):
    cpt = num_edges // (NW * chunk)
    assert cpt * NW * chunk == num_edges
    npw = npad // NS
    nvec = chunk // L
    mesh = plsc.VectorSubcoreMesh(core_axis_name="c", subcore_axis_name="s",
                                  num_cores=NC, num_subcores=NS)

    nhw = npw // 2

    def body(ei, h_hbm, par_hbm, num_out, den_out,
             h_v, par_v, src0, src1, dst0, dst1, e_v, eh_v,
             zero_v, num_sh, den_sh, sem_h, sem_ld0, sem_ld1, sem_sc):
        c = lax.axis_index("c")
        s = lax.axis_index("s")
        w = c * NS + s
        ebase = w * (cpt * chunk)
        srcs, dsts = [src0, src1], [dst0, dst1]
        lsems = [sem_ld0, sem_ld1]

        h_desc = pltpu.async_copy(h_hbm, h_v, sem_h)

        def zbody(j, _):
            zero_v[pl.ds(j * L, L)] = jnp.zeros((L,), jnp.float32)
            return 0

        lax.fori_loop(0, nhw // L, zbody, 0)
        for k in range(2):
            pltpu.sync_copy(zero_v, num_sh.at[pl.ds(s * npw + k * nhw, nhw)])
            pltpu.sync_copy(zero_v, den_sh.at[pl.ds(s * npw + k * nhw, nhw)])
        pltpu.sync_copy(par_hbm, par_v)
        h_desc.wait()
        plsc.subcore_barrier()
        ats = par_v[0]
        atd = par_v[1]
        cshift = par_v[2]

        def issue_loads(t, p):
            b = ebase + t * chunk
            return (pltpu.async_copy(ei.at[pl.ds(b, chunk)], srcs[p],
                                     lsems[p]),
                    pltpu.async_copy(ei.at[pl.ds(num_edges + b, chunk)],
                                     dsts[p], lsems[p]))

        ld = [None, None]
        sc_d = None
        ld[0] = issue_loads(0, 0)
        for t in range(cpt):
            p = t % 2
            q = 1 - p
            if sc_d is not None and t + 1 < cpt:
                for d_ in sc_d:
                    d_.wait()
                sc_d = None
            if t + 1 < cpt:
                ld[q] = issue_loads(t + 1, q)
            for d_ in ld[p]:
                d_.wait()
            if sc_d is not None:
                for d_ in sc_d:
                    d_.wait()
                sc_d = None
            src_v, dst_v = srcs[p], dsts[p]

            def vbody(j, _):
                sidx = src_v[pl.ds(j * L, L)]
                didx = dst_v[pl.ds(j * L, L)]
                hs = plsc.load_gather(h_v, [sidx])
                hd = plsc.load_gather(h_v, [didx])
                z = ats * hs + atd * hd
                v = jnp.where(z >= 0, z, NEG_SLOPE * z)
                e = jnp.exp(v - cshift)
                e_v[pl.ds(j * L, L)] = e
                eh_v[pl.ds(j * L, L)] = e * hs
                return 0

            lax.fori_loop(0, nvec, vbody, 0)
            sc_d = (pltpu.async_copy(e_v, den_sh.at[dst_v], sem_sc,
                                     add=True),
                    pltpu.async_copy(eh_v, num_sh.at[dst_v], sem_sc,
                                     add=True))
        for d_ in sc_d:
            d_.wait()
        plsc.subcore_barrier()
        for k in range(2):
            pltpu.sync_copy(num_sh.at[pl.ds(s * npw + k * nhw, nhw)], zero_v)
            pltpu.sync_copy(
                zero_v, num_out.at[pl.ds(c * npad + s * npw + k * nhw, nhw)])
            pltpu.sync_copy(den_sh.at[pl.ds(s * npw + k * nhw, nhw)], zero_v)
            pltpu.sync_copy(
                zero_v, den_out.at[pl.ds(c * npad + s * npw + k * nhw, nhw)])

    return pl.kernel(
        body,
        out_type=[jax.ShapeDtypeStruct((NC * npad,), jnp.float32),
                  jax.ShapeDtypeStruct((NC * npad,), jnp.float32)],
        mesh=mesh,
        compiler_params=pltpu.CompilerParams(needs_layout_passes=False),
        scratch_types=[
            pltpu.VMEM((npad,), jnp.float32),
            pltpu.VMEM((4, L), jnp.float32),
            pltpu.VMEM((chunk,), jnp.int32),
            pltpu.VMEM((chunk,), jnp.int32),
            pltpu.VMEM((chunk,), jnp.int32),
            pltpu.VMEM((chunk,), jnp.int32),
            pltpu.VMEM((chunk,), jnp.float32),
            pltpu.VMEM((chunk,), jnp.float32),
            pltpu.VMEM((npad // NS // 2,), jnp.float32),
            pltpu.VMEM_SHARED((npad,), jnp.float32),
            pltpu.VMEM_SHARED((npad,), jnp.float32),
            pltpu.SemaphoreType.DMA,
            pltpu.SemaphoreType.DMA,
            pltpu.SemaphoreType.DMA,
            pltpu.SemaphoreType.DMA,
        ],
    )


def _pass_c(par, x, h3, batch3, num4, den4, nrows, n, g):
    d = x.shape[1]

    nb = -(-nrows // BR)

    def body(par_ref, x_ref, h_ref, bat_ref, n0_ref, n1_ref, d0_ref, d1_ref,
             gx_ref, dacc):
        i = pl.program_id(0)
        atsum = par_ref[0]
        csh = par_ref[1]
        bb = par_ref[2]
        m2 = par_ref[3]
        h = h_ref[:, 0, :]
        n0 = n0_ref[0, :, 0, :]
        n1 = n1_ref[0, :, 0, :]
        d0 = d0_ref[0, :, 0, :]
        d1 = d1_ref[0, :, 0, :]
        z = atsum * h
        vv = jnp.where(z >= 0, z, NEG_SLOPE * z)
        es = jnp.exp(vv - csh)
        ntot = n0 + n1 + es * h
        dtot = d0 + d1 + es
        sval = ntot / dtot + bb
        e2 = jnp.exp(sval - m2)
        lane = lax.broadcasted_iota(jnp.int32, (BR, 128), 1)
        row = lax.broadcasted_iota(jnp.int32, (BR, 128), 0)
        nid = (i * BR + row) * 128 + lane
        e2 = jnp.where(nid < n, e2, 0.0)
        bat = bat_ref[:, 0, :]
        gid = lax.broadcasted_iota(jnp.int32, (g, 1), 0)
        xrow = lax.broadcasted_iota(jnp.int32, (BR * 128, 1), 0)
        xb = jnp.where(i * BR * 128 + xrow < n, x_ref[...], 0.0)
        part = None
        partd = None
        for r in range(BR):
            onehot_t = jnp.where(bat[r:r + 1, :] == gid, e2[r:r + 1, :], 0.0)
            pr = lax.dot_general(onehot_t, xb[r * 128:(r + 1) * 128, :],
                                 (((1,), (0,)), ((), ())),
                                 preferred_element_type=jnp.float32)
            dr = jnp.sum(onehot_t, axis=1, keepdims=True)
            part = pr if part is None else part + pr
            partd = dr if partd is None else partd + dr

        @pl.when(i == 0)
        def _():
            gx_ref[...] = part
            dacc[...] = partd

        @pl.when(i > 0)
        def _():
            gx_ref[...] += part
            dacc[...] += partd

        @pl.when(i == nb - 1)
        def _():
            gx_ref[...] = gx_ref[...] / (dacc[...] + 1e-16)

    return pl.pallas_call(
        body,
        grid=(nb,),
        in_specs=[pl.BlockSpec(memory_space=pltpu.SMEM),
                  pl.BlockSpec((BR * 128, d), lambda i: (i, 0)),
                  pl.BlockSpec((BR, 1, 128), lambda i: (i, 0, 0)),
                  pl.BlockSpec((BR, 1, 128), lambda i: (i, 0, 0)),
                  pl.BlockSpec((1, BR, 1, 128), lambda i: (0, i, 0, 0)),
                  pl.BlockSpec((1, BR, 1, 128), lambda i: (1, i, 0, 0)),
                  pl.BlockSpec((1, BR, 1, 128), lambda i: (0, i, 0, 0)),
                  pl.BlockSpec((1, BR, 1, 128), lambda i: (1, i, 0, 0))],
        out_specs=pl.BlockSpec((g, d), lambda i: (0, 0)),
        out_shape=jax.ShapeDtypeStruct((g, d), jnp.float32),
        scratch_shapes=[pltpu.VMEM((g, 1), jnp.float32)],
    )(par, x, h3, batch3, num4, num4, den4, den4)


def kernel(x, edge_index, batch, W, b, att_src, att_dst):
    n, d = x.shape
    num_edges = edge_index.shape[1]
    g = NUM_GRAPHS
    nrows = -(-n // 128)
    npad = nrows * 128
    chunk = 2000

    w_row = W.reshape(1, d)
    h3, hmax, hmin = _pass_a(x, w_row, nrows, n)
    hmax_s = hmax[0]
    hmin_s = hmin[0]
    mas = jnp.where(att_src[0] >= 0, att_src[0] * hmax_s, att_src[0] * hmin_s)
    mad = jnp.where(att_dst[0] >= 0, att_dst[0] * hmax_s, att_dst[0] * hmin_s)
    amax = mas + mad
    cshift = jnp.where(amax >= 0, amax, NEG_SLOPE * amax)

    par_sc = jnp.stack([
        jnp.full((L,), att_src[0], jnp.float32),
        jnp.full((L,), att_dst[0], jnp.float32),
        jnp.full((L,), cshift, jnp.float32),
        jnp.zeros((L,), jnp.float32),
    ])
    h_flat = h3.reshape(npad)
    num2, den2 = _make_edge_kernel(num_edges, npad, chunk)(
        edge_index.reshape(2 * num_edges), h_flat, par_sc)

    m2 = hmax_s + b[0]
    par_tc = jnp.stack([att_src[0] + att_dst[0], cshift, b[0], m2])
    batch_p = jnp.concatenate(
        [batch, jnp.full((npad - n,), g, jnp.int32)]).reshape(nrows, 1, 128)
    num4 = num2.reshape(NC, nrows, 1, 128)
    den4 = den2.reshape(NC, nrows, 1, 128)
    return _pass_c(par_tc, x, h3, batch_p, num4, den4, nrows, n, g)


# speedup vs baseline: 221.7417x; 1.0934x over previous
import functools

import jax
import jax.numpy as jnp
from jax import lax
from jax.experimental import pallas as pl
from jax.experimental.pallas import tpu as pltpu
from jax.experimental.pallas import tpu_sc as plsc

NC, NS, L = 2, 16, 16
NW = NC * NS
NEG_SLOPE = 0.2
NUM_GRAPHS = 512


BR = 8


def _pass_a(x, w_row, nrows, n, br):
    d = x.shape[1]
    nb = -(-nrows // br)

    def body(wt_ref, x_ref, h_ref, hmax_ref, hmin_ref):
        i = pl.program_id(0)
        xb = x_ref[...]
        wt = wt_ref[...]
        lane = lax.broadcasted_iota(jnp.int32, (1, 128), 1)
        bmax = None
        for r in range(br):
            hrow = lax.dot_general(wt, xb[r * 128:(r + 1) * 128, :],
                                   (((1,), (1,)), ((), ())),
                                   preferred_element_type=jnp.float32)
            valid = ((i * br + r) * 128 + lane) < n
            hrow = jnp.where(valid, hrow, 0.0)
            h_ref[r] = hrow
            rmax = jnp.max(hrow)
            rmin = jnp.min(hrow)
            bmax = rmax if bmax is None else jnp.maximum(bmax, rmax)
            bmin = rmin if r == 0 else jnp.minimum(bmin, rmin)

        @pl.when(i == 0)
        def _():
            hmax_ref[0] = bmax
            hmin_ref[0] = bmin

        @pl.when(i > 0)
        def _():
            hmax_ref[0] = jnp.maximum(hmax_ref[0], bmax)
            hmin_ref[0] = jnp.minimum(hmin_ref[0], bmin)

    return pl.pallas_call(
        body,
        grid=(nb,),
        in_specs=[pl.BlockSpec((1, d), lambda i: (0, 0)),
                  pl.BlockSpec((br * 128, d), lambda i: (i, 0))],
        out_specs=[pl.BlockSpec((br, 1, 128), lambda i: (i, 0, 0)),
                   pl.BlockSpec(memory_space=pltpu.SMEM),
                   pl.BlockSpec(memory_space=pltpu.SMEM)],
        out_shape=[jax.ShapeDtypeStruct((nrows, 1, 128), jnp.float32),
                   jax.ShapeDtypeStruct((1,), jnp.float32),
                   jax.ShapeDtypeStruct((1,), jnp.float32)],
    )(w_row, x)


def _make_edge_kernel(num_edges, npad, chunk):
    cpt = num_edges // (NW * chunk)
    assert cpt * NW * chunk == num_edges
    npw = npad // NS
    nvec = chunk // L
    mesh = plsc.VectorSubcoreMesh(core_axis_name="c", subcore_axis_name="s",
                                  num_cores=NC, num_subcores=NS)

    nhw = npw // 2

    def body(ei, h_hbm, par_hbm, num_out, den_out,
             h_v, par_v, src0, src1, dst0, dst1, e_v, eh_v,
             zero_v, num_sh, den_sh, sem_h, sem_ld0, sem_ld1, sem_sc):
        c = lax.axis_index("c")
        s = lax.axis_index("s")
        w = c * NS + s
        ebase = w * (cpt * chunk)
        srcs, dsts = [src0, src1], [dst0, dst1]
        lsems = [sem_ld0, sem_ld1]

        h_desc = pltpu.async_copy(h_hbm, h_v, sem_h)

        def zbody(j, _):
            zero_v[pl.ds(j * L, L)] = jnp.zeros((L,), jnp.float32)
            return 0

        lax.fori_loop(0, nhw // L, zbody, 0)
        if nhw % L:
            zero_v[pl.ds(nhw - L, L)] = jnp.zeros((L,), jnp.float32)
        for k in range(2):
            pltpu.sync_copy(zero_v, num_sh.at[pl.ds(s * npw + k * nhw, nhw)])
            pltpu.sync_copy(zero_v, den_sh.at[pl.ds(s * npw + k * nhw, nhw)])
        pltpu.sync_copy(par_hbm, par_v)
        h_desc.wait()
        plsc.subcore_barrier()
        ats = par_v[0]
        atd = par_v[1]
        cshift = par_v[2]

        def issue_loads(t, p):
            b = ebase + t * chunk
            return (pltpu.async_copy(ei.at[pl.ds(b, chunk)], srcs[p],
                                     lsems[p]),
                    pltpu.async_copy(ei.at[pl.ds(num_edges + b, chunk)],
                                     dsts[p], lsems[p]))

        ld = [None, None]
        sc_d = None
        ld[0] = issue_loads(0, 0)
        for t in range(cpt):
            p = t % 2
            q = 1 - p
            if sc_d is not None and t + 1 < cpt:
                for d_ in sc_d:
                    d_.wait()
                sc_d = None
            if t + 1 < cpt:
                ld[q] = issue_loads(t + 1, q)
            for d_ in ld[p]:
                d_.wait()
            if sc_d is not None:
                for d_ in sc_d:
                    d_.wait()
                sc_d = None
            src_v, dst_v = srcs[p], dsts[p]

            def vbody(j, _):
                sidx = src_v[pl.ds(j * L, L)]
                didx = dst_v[pl.ds(j * L, L)]
                hs = plsc.load_gather(h_v, [sidx])
                hd = plsc.load_gather(h_v, [didx])
                z = ats * hs + atd * hd
                v = jnp.where(z >= 0, z, NEG_SLOPE * z)
                e = jnp.exp(v - cshift)
                e_v[pl.ds(j * L, L)] = e
                eh_v[pl.ds(j * L, L)] = e * hs
                return 0

            lax.fori_loop(0, nvec, vbody, 0)
            sc_d = (pltpu.async_copy(e_v, den_sh.at[dst_v], sem_sc,
                                     add=True),
                    pltpu.async_copy(eh_v, num_sh.at[dst_v], sem_sc,
                                     add=True))
        for d_ in sc_d:
            d_.wait()
        plsc.subcore_barrier()
        for k in range(2):
            pltpu.sync_copy(num_sh.at[pl.ds(s * npw + k * nhw, nhw)], zero_v)
            pltpu.sync_copy(
                zero_v, num_out.at[pl.ds(c * npad + s * npw + k * nhw, nhw)])
            pltpu.sync_copy(den_sh.at[pl.ds(s * npw + k * nhw, nhw)], zero_v)
            pltpu.sync_copy(
                zero_v, den_out.at[pl.ds(c * npad + s * npw + k * nhw, nhw)])

    return pl.kernel(
        body,
        out_type=[jax.ShapeDtypeStruct((NC * npad,), jnp.float32),
                  jax.ShapeDtypeStruct((NC * npad,), jnp.float32)],
        mesh=mesh,
        compiler_params=pltpu.CompilerParams(needs_layout_passes=False),
        scratch_types=[
            pltpu.VMEM((npad,), jnp.float32),
            pltpu.VMEM((4, L), jnp.float32),
            pltpu.VMEM((chunk,), jnp.int32),
            pltpu.VMEM((chunk,), jnp.int32),
            pltpu.VMEM((chunk,), jnp.int32),
            pltpu.VMEM((chunk,), jnp.int32),
            pltpu.VMEM((chunk,), jnp.float32),
            pltpu.VMEM((chunk,), jnp.float32),
            pltpu.VMEM((npad // NS // 2,), jnp.float32),
            pltpu.VMEM_SHARED((npad,), jnp.float32),
            pltpu.VMEM_SHARED((npad,), jnp.float32),
            pltpu.SemaphoreType.DMA,
            pltpu.SemaphoreType.DMA,
            pltpu.SemaphoreType.DMA,
            pltpu.SemaphoreType.DMA,
        ],
    )


def _pass_c(par, x, h3, batch3, num4, den4, nrows, n, g):
    d = x.shape[1]

    nb = -(-nrows // BR)

    def body(par_ref, x_ref, h_ref, bat_ref, n0_ref, n1_ref, d0_ref, d1_ref,
             gx_ref, dacc):
        i = pl.program_id(0)
        atsum = par_ref[0]
        csh = par_ref[1]
        bb = par_ref[2]
        m2 = par_ref[3]
        h = h_ref[:, 0, :]
        n0 = n0_ref[0, :, 0, :]
        n1 = n1_ref[0, :, 0, :]
        d0 = d0_ref[0, :, 0, :]
        d1 = d1_ref[0, :, 0, :]
        z = atsum * h
        vv = jnp.where(z >= 0, z, NEG_SLOPE * z)
        es = jnp.exp(vv - csh)
        ntot = n0 + n1 + es * h
        dtot = d0 + d1 + es
        sval = ntot / dtot + bb
        e2 = jnp.exp(sval - m2)
        lane = lax.broadcasted_iota(jnp.int32, (BR, 128), 1)
        row = lax.broadcasted_iota(jnp.int32, (BR, 128), 0)
        nid = (i * BR + row) * 128 + lane
        e2 = jnp.where(nid < n, e2, 0.0)
        bat = bat_ref[:, 0, :]
        gid = lax.broadcasted_iota(jnp.int32, (g, 1), 0)
        xrow = lax.broadcasted_iota(jnp.int32, (BR * 128, 1), 0)
        xb = jnp.where(i * BR * 128 + xrow < n, x_ref[...], 0.0)
        part = None
        partd = None
        for r in range(BR):
            onehot_t = jnp.where(bat[r:r + 1, :] == gid, e2[r:r + 1, :], 0.0)
            pr = lax.dot_general(onehot_t, xb[r * 128:(r + 1) * 128, :],
                                 (((1,), (0,)), ((), ())),
                                 preferred_element_type=jnp.float32)
            dr = jnp.sum(onehot_t, axis=1, keepdims=True)
            part = pr if part is None else part + pr
            partd = dr if partd is None else partd + dr

        @pl.when(i == 0)
        def _():
            gx_ref[...] = part
            dacc[...] = partd

        @pl.when(i > 0)
        def _():
            gx_ref[...] += part
            dacc[...] += partd

        @pl.when(i == nb - 1)
        def _():
            gx_ref[...] = gx_ref[...] / (dacc[...] + 1e-16)

    return pl.pallas_call(
        body,
        grid=(nb,),
        in_specs=[pl.BlockSpec(memory_space=pltpu.SMEM),
                  pl.BlockSpec((BR * 128, d), lambda i: (i, 0)),
                  pl.BlockSpec((BR, 1, 128), lambda i: (i, 0, 0)),
                  pl.BlockSpec((BR, 1, 128), lambda i: (i, 0, 0)),
                  pl.BlockSpec((1, BR, 1, 128), lambda i: (0, i, 0, 0)),
                  pl.BlockSpec((1, BR, 1, 128), lambda i: (1, i, 0, 0)),
                  pl.BlockSpec((1, BR, 1, 128), lambda i: (0, i, 0, 0)),
                  pl.BlockSpec((1, BR, 1, 128), lambda i: (1, i, 0, 0))],
        out_specs=pl.BlockSpec((g, d), lambda i: (0, 0)),
        out_shape=jax.ShapeDtypeStruct((g, d), jnp.float32),
        scratch_shapes=[pltpu.VMEM((g, 1), jnp.float32)],
    )(par, x, h3, batch3, num4, num4, den4, den4)


def kernel(x, edge_index, batch, W, b, att_src, att_dst):
    n, d = x.shape
    num_edges = edge_index.shape[1]
    g = NUM_GRAPHS
    nrows = -(-n // 128)
    npad = nrows * 128
    chunk = 2000

    w_row = W.reshape(1, d)
    h3, hmax, hmin = _pass_a(x, w_row, nrows, n, 16)
    hmax_s = hmax[0]
    hmin_s = hmin[0]
    mas = jnp.where(att_src[0] >= 0, att_src[0] * hmax_s, att_src[0] * hmin_s)
    mad = jnp.where(att_dst[0] >= 0, att_dst[0] * hmax_s, att_dst[0] * hmin_s)
    amax = mas + mad
    cshift = jnp.where(amax >= 0, amax, NEG_SLOPE * amax)

    par_sc = jnp.stack([
        jnp.full((L,), att_src[0], jnp.float32),
        jnp.full((L,), att_dst[0], jnp.float32),
        jnp.full((L,), cshift, jnp.float32),
        jnp.zeros((L,), jnp.float32),
    ])
    h_flat = h3.reshape(npad)
    num2, den2 = _make_edge_kernel(num_edges, npad, chunk)(
        edge_index.reshape(2 * num_edges), h_flat, par_sc)

    m2 = hmax_s + b[0]
    par_tc = jnp.stack([att_src[0] + att_dst[0], cshift, b[0], m2])
    batch_p = jnp.concatenate(
        [batch, jnp.full((npad - n,), g, jnp.int32)]).reshape(nrows, 1, 128)
    num4 = num2.reshape(NC, nrows, 1, 128)
    den4 = den2.reshape(NC, nrows, 1, 128)
    return _pass_c(par_tc, x, h3, batch_p, num4, den4, nrows, n, g)
